# baseline (speedup 1.0000x reference)
import functools

import jax
import jax.numpy as jnp
from jax.experimental import pallas as pl

B, N, IN = 4, 2048, 1024
R = 256
E = 8
K = 2
OUT = 1024
TEMP = 1.0

TBLK = 512


def _fused_kernel(x_ref, wd_ref, wg_ref, wcat_ref, be_ref, wug_ref, s_ref,
                  out_ref):
    i = pl.program_id(0)
    xb = x_ref[...]
    hp = jax.lax.dot_general(xb, wd_ref[...], (((1,), (1,)), ((), ())),
                             preferred_element_type=jnp.float32)
    h = 0.5 * hp * (1.0 + jax.lax.erf(hp * 0.7071067811865476))

    lt = jax.lax.dot_general(wg_ref[...], h, (((1,), (1,)), ((), ())),
                             preferred_element_type=jnp.float32)
    lmax = jnp.max(lt, axis=0, keepdims=True)
    u = jnp.exp((lt - lmax) / TEMP)
    eidx = jax.lax.broadcasted_iota(jnp.int32, u.shape, 0)
    m1 = jnp.max(u, axis=0, keepdims=True)
    idx1 = jnp.min(jnp.where(u == m1, eidx, E), axis=0, keepdims=True)
    sel1 = eidx == idx1
    u2 = jnp.where(sel1, -jnp.inf, u)
    m2 = jnp.max(u2, axis=0, keepdims=True)
    idx2 = jnp.min(jnp.where(u2 == m2, eidx, E), axis=0, keepdims=True)
    sel2 = eidx == idx2
    denom = m1 + m2
    wt = (jnp.where(sel1, m1, 0.0) + jnp.where(sel2, m2, 0.0)) / denom

    col = jax.lax.broadcasted_iota(jnp.int32, u.shape, 1) + i * TBLK
    wt = jnp.where((col % N) >= 5, wt, 0.0)

    wtok = jax.lax.dot_general(wt, s_ref[...], (((0,), (0,)), ((), ())),
                               preferred_element_type=jnp.float32)
    wtok_b = wtok.astype(jnp.bfloat16)
    hb = h.astype(jnp.bfloat16)
    hcat = jnp.concatenate([hb * wtok_b[:, e:e + 1] for e in range(E)], axis=1)
    comb = jax.lax.dot_general(hcat, wcat_ref[...], (((1,), (0,)), ((), ())),
                               preferred_element_type=jnp.float32)
    acc = jax.lax.dot_general(wt, be_ref[...], (((0,), (0,)), ((), ())),
                              preferred_element_type=jnp.float32)

    tok = h + comb + acc
    out_ref[...] = jax.lax.dot_general(tok, wug_ref[...],
                                       (((1,), (1,)), ((), ())),
                                       preferred_element_type=jnp.float32)


@functools.partial(jax.jit, static_argnames=())
def kernel(x, Wd, Wg, We, be, Wu, gamma):
    xf = x.reshape(B * N, IN)
    wcat = jnp.transpose(We, (0, 2, 1)).reshape(E * R, R).astype(jnp.bfloat16)
    wug = Wu * gamma[:, None]
    s = jnp.eye(E, dtype=jnp.float32)
    grid = (B * N // TBLK,)
    out = pl.pallas_call(
        _fused_kernel,
        grid=grid,
        in_specs=[
            pl.BlockSpec((TBLK, IN), lambda i: (i, 0)),
            pl.BlockSpec((R, IN), lambda i: (0, 0)),
            pl.BlockSpec((E, R), lambda i: (0, 0)),
            pl.BlockSpec((E * R, R), lambda i: (0, 0)),
            pl.BlockSpec((E, R), lambda i: (0, 0)),
            pl.BlockSpec((OUT, R), lambda i: (0, 0)),
            pl.BlockSpec((E, E), lambda i: (0, 0)),
        ],
        out_specs=pl.BlockSpec((TBLK, OUT), lambda i: (i, 0)),
        out_shape=jax.ShapeDtypeStruct((B * N, OUT), jnp.float32),
    )(xf, Wd, Wg, wcat, be, wug, s)
    return out.reshape(B, N, OUT)


# speedup vs baseline: 1.4853x; 1.4853x over previous
import functools

import jax
import jax.numpy as jnp
from jax.experimental import pallas as pl

B, N, IN = 4, 2048, 1024
R = 256
E = 8
K = 2
OUT = 1024
TEMP = 1.0

TBLK = 1024


def _fused_kernel(x_ref, wd_ref, wg_ref, we_ref, be_ref, wug_ref, out_ref):
    i = pl.program_id(0)
    xb = x_ref[...]
    hpt = jax.lax.dot_general(wd_ref[...], xb, (((1,), (1,)), ((), ())),
                              preferred_element_type=jnp.float32)
    ht = 0.5 * hpt * (1.0 + jax.lax.erf(hpt * 0.7071067811865476))

    lt = jax.lax.dot_general(wg_ref[...], ht, (((1,), (0,)), ((), ())),
                             preferred_element_type=jnp.float32)
    lmax = jnp.max(lt, axis=0, keepdims=True)
    u = jnp.exp((lt - lmax) / TEMP)
    eidx = jax.lax.broadcasted_iota(jnp.int32, u.shape, 0)
    m1 = jnp.max(u, axis=0, keepdims=True)
    idx1 = jnp.min(jnp.where(u == m1, eidx, E), axis=0, keepdims=True)
    sel1 = eidx == idx1
    u2 = jnp.where(sel1, -jnp.inf, u)
    m2 = jnp.max(u2, axis=0, keepdims=True)
    idx2 = jnp.min(jnp.where(u2 == m2, eidx, E), axis=0, keepdims=True)
    sel2 = eidx == idx2
    denom = m1 + m2
    wt = (jnp.where(sel1, m1, 0.0) + jnp.where(sel2, m2, 0.0)) / denom

    col = jax.lax.broadcasted_iota(jnp.int32, u.shape, 1) + i * TBLK
    wt = jnp.where((col % N) >= 5, wt, 0.0)

    gt = jax.lax.dot_general(we_ref[...], ht, (((1,), (0,)), ((), ())),
                             preferred_element_type=jnp.float32)
    acc = jax.lax.dot_general(be_ref[...], wt, (((0,), (0,)), ((), ())),
                              preferred_element_type=jnp.float32)
    for e in range(E):
        acc = acc + gt[e * R:(e + 1) * R, :] * wt[e:e + 1, :]

    tokt = ht + acc
    out_ref[...] = jax.lax.dot_general(tokt, wug_ref[...],
                                       (((0,), (1,)), ((), ())),
                                       preferred_element_type=jnp.float32)


@functools.partial(jax.jit, static_argnames=())
def kernel(x, Wd, Wg, We, be, Wu, gamma):
    xf = x.reshape(B * N, IN)
    westack = We.reshape(E * R, R)
    wug = Wu * gamma[:, None]
    grid = (B * N // TBLK,)
    out = pl.pallas_call(
        _fused_kernel,
        grid=grid,
        in_specs=[
            pl.BlockSpec((TBLK, IN), lambda i: (i, 0)),
            pl.BlockSpec((R, IN), lambda i: (0, 0)),
            pl.BlockSpec((E, R), lambda i: (0, 0)),
            pl.BlockSpec((E * R, R), lambda i: (0, 0)),
            pl.BlockSpec((E, R), lambda i: (0, 0)),
            pl.BlockSpec((OUT, R), lambda i: (0, 0)),
        ],
        out_specs=pl.BlockSpec((TBLK, OUT), lambda i: (i, 0)),
        out_shape=jax.ShapeDtypeStruct((B * N, OUT), jnp.float32),
    )(xf, Wd, Wg, westack, be, wug)
    return out.reshape(B, N, OUT)
